# baseline (speedup 1.0000x reference)
import functools

import jax
import jax.numpy as jnp
from jax import lax
from jax.experimental import pallas as pl
from jax.experimental.pallas import tpu as pltpu
from jax.experimental.pallas import tpu_sc as plsc

_NC = 2
_NS = 16
_L = 16


def _make_sc_mask(E, P, NMAX):
    e_w = E // (_NC * _NS)
    assert e_w % _L == 0 and e_w % 128 == 0
    n_chunk = e_w // 128
    p_s = P // _NS
    assert p_s % 8 == 0

    mesh = plsc.VectorSubcoreMesh(
        core_axis_name="c", subcore_axis_name="s",
        num_cores=_NC, num_subcores=_NS)

    @functools.partial(
        pl.kernel,
        out_type=jax.ShapeDtypeStruct((_NC, P), jnp.float32),
        mesh=mesh,
        scratch_types=[
            pltpu.VMEM((e_w,), jnp.int32),
            pltpu.VMEM((e_w,), jnp.int32),
            pltpu.VMEM((n_chunk, 128), jnp.int32),
            pltpu.VMEM((128,), jnp.float32),
            pltpu.VMEM((p_s,), jnp.float32),
            pltpu.VMEM_SHARED((P,), jnp.float32),
        ],
    )
    def sc_mask(ring_hbm, out_hbm, src_v, dst_v, idx_v, ones_v, zero_v, mask_sh):
        c = lax.axis_index("c")
        s = lax.axis_index("s")
        base = (c * _NS + s) * e_w
        i32_0 = c * 0
        i32_1 = i32_0 + 1

        pltpu.sync_copy(ring_hbm.at[i32_0, pl.ds(base, e_w)], src_v)
        pltpu.sync_copy(ring_hbm.at[i32_1, pl.ds(base, e_w)], dst_v)

        i32_L = i32_0 + _L

        def zfill(i, carry):
            zero_v[pl.ds(i * i32_L, _L)] = jnp.zeros((_L,), jnp.float32)
            return carry
        lax.fori_loop(jnp.int32(0), jnp.int32(p_s // _L), zfill, 0)

        for t in range(128 // _L):
            ones_v[pl.ds(t * _L, _L)] = jnp.full((_L,), 1.0, jnp.float32)

        for j in range(n_chunk):
            for t in range(128 // _L):
                off = j * 128 + t * _L
                sv = src_v[pl.ds(off, _L)]
                dv = dst_v[pl.ds(off, _L)]
                idx_v[i32_0 + j, pl.ds(t * _L, _L)] = sv * NMAX + (dv & (NMAX - 1))

        pltpu.sync_copy(zero_v, mask_sh.at[pl.ds(s * p_s, p_s)])
        plsc.subcore_barrier()
        for j in range(n_chunk):
            pltpu.sync_copy(ones_v, mask_sh.at[idx_v.at[i32_0 + j]], add=True)
        plsc.subcore_barrier()
        pltpu.sync_copy(mask_sh.at[pl.ds(s * p_s, p_s)],
                        out_hbm.at[c, pl.ds(s * p_s, p_s)])

    return sc_mask


def _dense_body(m_ref, w_ref, x_ref, o_ref):
    m = m_ref[...]
    x = x_ref[...]
    w1 = w_ref[1, :]
    factor = jnp.minimum(m[0] + m[1], 1.0)
    o_ref[...] = x + factor[:, :, None] * w1[None, None, :]


def _dense_add(x, m, w, block_g=32):
    R, NMAX, EMB = x.shape
    grid = (R // block_g,)
    return pl.pallas_call(
        _dense_body,
        grid=grid,
        in_specs=[
            pl.BlockSpec((2, block_g, NMAX), lambda i: (i * 0, i, i * 0)),
            pl.BlockSpec((2, EMB), lambda i: (i * 0, i * 0)),
            pl.BlockSpec((block_g, NMAX, EMB), lambda i: (i, i * 0, i * 0)),
        ],
        out_specs=pl.BlockSpec((block_g, NMAX, EMB), lambda i: (i, i * 0, i * 0)),
        out_shape=jax.ShapeDtypeStruct((R, NMAX, EMB), jnp.float32),
        compiler_params=pltpu.CompilerParams(
            dimension_semantics=("parallel",),
        ),
    )(m, w, x)


def kernel(edge_dense, ring_index, batch, weight):
    B, NMAX, _, EMB = edge_dense.shape
    E = ring_index.shape[1]
    R = B * NMAX
    P = R * NMAX

    ring32 = ring_index.astype(jnp.int32)
    mask2 = _make_sc_mask(E, P, NMAX)(ring32)
    m = mask2.reshape(2, R, NMAX)

    x = edge_dense.reshape(R, NMAX, EMB)
    w = weight.astype(jnp.float32)
    out = _dense_add(x, m, w)
    return out.reshape(B, NMAX, NMAX, EMB)


# speedup vs baseline: 1.1078x; 1.1078x over previous
import functools

import jax
import jax.numpy as jnp
from jax import lax
from jax.experimental import pallas as pl
from jax.experimental.pallas import tpu as pltpu
from jax.experimental.pallas import tpu_sc as plsc

_NC = 2
_NS = 16
_L = 16


def _make_sc_mask(E, P, NMAX):
    e_w = E // (_NC * _NS)
    assert e_w % _L == 0 and e_w % 128 == 0
    n_chunk = e_w // 128
    p_s = P // _NS
    assert p_s % 8 == 0

    mesh = plsc.VectorSubcoreMesh(
        core_axis_name="c", subcore_axis_name="s",
        num_cores=_NC, num_subcores=_NS)

    @functools.partial(
        pl.kernel,
        out_type=jax.ShapeDtypeStruct((_NC, P), jnp.float32),
        mesh=mesh,
        scratch_types=[
            pltpu.VMEM((e_w,), jnp.int32),
            pltpu.VMEM((e_w,), jnp.int32),
            pltpu.VMEM((n_chunk, 128), jnp.int32),
            pltpu.VMEM((128,), jnp.float32),
            pltpu.VMEM((p_s,), jnp.float32),
            pltpu.VMEM_SHARED((P,), jnp.float32),
        ],
    )
    def sc_mask(ring_hbm, out_hbm, src_v, dst_v, idx_v, ones_v, zero_v, mask_sh):
        c = lax.axis_index("c")
        s = lax.axis_index("s")
        base = (c * _NS + s) * e_w
        i32_0 = c * 0
        i32_1 = i32_0 + 1

        pltpu.sync_copy(ring_hbm.at[i32_0, pl.ds(base, e_w)], src_v)
        pltpu.sync_copy(ring_hbm.at[i32_1, pl.ds(base, e_w)], dst_v)

        i32_L = i32_0 + _L

        def zfill(i, carry):
            zero_v[pl.ds(i * i32_L, _L)] = jnp.zeros((_L,), jnp.float32)
            return carry
        lax.fori_loop(jnp.int32(0), jnp.int32(p_s // _L), zfill, 0)

        for t in range(128 // _L):
            ones_v[pl.ds(t * _L, _L)] = jnp.full((_L,), 1.0, jnp.float32)

        for j in range(n_chunk):
            for t in range(128 // _L):
                off = j * 128 + t * _L
                sv = src_v[pl.ds(off, _L)]
                dv = dst_v[pl.ds(off, _L)]
                idx_v[i32_0 + j, pl.ds(t * _L, _L)] = sv * NMAX + (dv & (NMAX - 1))

        pltpu.sync_copy(zero_v, mask_sh.at[pl.ds(s * p_s, p_s)])
        plsc.subcore_barrier()
        for j in range(n_chunk):
            pltpu.sync_copy(ones_v, mask_sh.at[idx_v.at[i32_0 + j]], add=True)
        plsc.subcore_barrier()
        pltpu.sync_copy(mask_sh.at[pl.ds(s * p_s, p_s)],
                        out_hbm.at[c, pl.ds(s * p_s, p_s)])

    return sc_mask


def _dense_body(m_ref, w_ref, x_ref, o_ref):
    m = m_ref[...]
    x = x_ref[...]
    w1 = w_ref[1, :]
    factor = jnp.minimum(m[0] + m[1], 1.0)
    o_ref[...] = x + factor[:, :, None] * w1[None, None, :]


def _dense_add(x, m, w, block_g=128):
    R, NMAX, EMB = x.shape
    grid = (R // block_g,)
    return pl.pallas_call(
        _dense_body,
        grid=grid,
        in_specs=[
            pl.BlockSpec((2, block_g, NMAX), lambda i: (i * 0, i, i * 0)),
            pl.BlockSpec((2, EMB), lambda i: (i * 0, i * 0)),
            pl.BlockSpec((block_g, NMAX, EMB), lambda i: (i, i * 0, i * 0)),
        ],
        out_specs=pl.BlockSpec((block_g, NMAX, EMB), lambda i: (i, i * 0, i * 0)),
        out_shape=jax.ShapeDtypeStruct((R, NMAX, EMB), jnp.float32),
        compiler_params=pltpu.CompilerParams(
            dimension_semantics=("parallel",),
        ),
    )(m, w, x)


def kernel(edge_dense, ring_index, batch, weight):
    B, NMAX, _, EMB = edge_dense.shape
    E = ring_index.shape[1]
    R = B * NMAX
    P = R * NMAX

    ring32 = ring_index.astype(jnp.int32)
    mask2 = _make_sc_mask(E, P, NMAX)(ring32)
    m = mask2.reshape(2, R, NMAX)

    x = edge_dense.reshape(R, NMAX, EMB)
    w = weight.astype(jnp.float32)
    out = _dense_add(x, m, w)
    return out.reshape(B, NMAX, NMAX, EMB)


# speedup vs baseline: 1.1350x; 1.0246x over previous
import functools

import jax
import jax.numpy as jnp
from jax import lax
from jax.experimental import pallas as pl
from jax.experimental.pallas import tpu as pltpu
from jax.experimental.pallas import tpu_sc as plsc

_NC = 2
_NS = 16
_L = 16


def _make_sc_mask(E, P, NMAX):
    e_w = E // (_NC * _NS)
    assert e_w % _L == 0 and e_w % 128 == 0
    n_chunk = e_w // 128
    p_s = P // _NS
    assert p_s % 8 == 0

    mesh = plsc.VectorSubcoreMesh(
        core_axis_name="c", subcore_axis_name="s",
        num_cores=_NC, num_subcores=_NS)

    @functools.partial(
        pl.kernel,
        out_type=jax.ShapeDtypeStruct((_NC, P), jnp.float32),
        mesh=mesh,
        scratch_types=[
            pltpu.VMEM((e_w,), jnp.int32),
            pltpu.VMEM((e_w,), jnp.int32),
            pltpu.VMEM((n_chunk, 128), jnp.int32),
            pltpu.VMEM((128,), jnp.float32),
            pltpu.VMEM((p_s,), jnp.float32),
            pltpu.VMEM_SHARED((P,), jnp.float32),
            pltpu.SemaphoreType.DMA,
            pltpu.SemaphoreType.DMA,
        ],
    )
    def sc_mask(ring_hbm, out_hbm, src_v, dst_v, idx_v, ones_v, zero_v,
                mask_sh, in_sem, sc_sem):
        c = lax.axis_index("c")
        s = lax.axis_index("s")
        base = (c * _NS + s) * e_w
        i32_0 = c * 0
        i32_1 = i32_0 + 1

        ld_src = pltpu.async_copy(ring_hbm.at[i32_0, pl.ds(base, e_w)], src_v, in_sem)
        ld_dst = pltpu.async_copy(ring_hbm.at[i32_1, pl.ds(base, e_w)], dst_v, in_sem)

        i32_L = i32_0 + _L

        def zfill(i, carry):
            zero_v[pl.ds(i * i32_L, _L)] = jnp.zeros((_L,), jnp.float32)
            return carry
        lax.fori_loop(jnp.int32(0), jnp.int32(p_s // _L), zfill, 0)

        for t in range(128 // _L):
            ones_v[pl.ds(t * _L, _L)] = jnp.full((_L,), 1.0, jnp.float32)

        zero_cp = pltpu.async_copy(zero_v, mask_sh.at[pl.ds(s * p_s, p_s)], sc_sem)

        ld_src.wait()
        ld_dst.wait()

        for j in range(n_chunk):
            for t in range(128 // _L):
                off = j * 128 + t * _L
                sv = src_v[pl.ds(off, _L)]
                dv = dst_v[pl.ds(off, _L)]
                idx_v[i32_0 + j, pl.ds(t * _L, _L)] = sv * NMAX + (dv & (NMAX - 1))

        zero_cp.wait()
        plsc.subcore_barrier()
        descs = [pltpu.async_copy(ones_v, mask_sh.at[idx_v.at[i32_0 + j]],
                                  sc_sem, add=True)
                 for j in range(n_chunk)]
        for d in descs:
            d.wait()
        plsc.subcore_barrier()
        pltpu.sync_copy(mask_sh.at[pl.ds(s * p_s, p_s)],
                        out_hbm.at[c, pl.ds(s * p_s, p_s)])

    return sc_mask


def _dense_body(m_ref, w_ref, x_ref, o_ref):
    m = m_ref[...]
    x = x_ref[...]
    w1 = w_ref[1, :]
    factor = jnp.minimum(m[0] + m[1], 1.0)
    o_ref[...] = x + factor[:, :, None] * w1[None, None, :]


def _dense_add(x, m, w, block_g=128):
    R, NMAX, EMB = x.shape
    grid = (R // block_g,)
    return pl.pallas_call(
        _dense_body,
        grid=grid,
        in_specs=[
            pl.BlockSpec((2, block_g, NMAX), lambda i: (i * 0, i, i * 0)),
            pl.BlockSpec((2, EMB), lambda i: (i * 0, i * 0)),
            pl.BlockSpec((block_g, NMAX, EMB), lambda i: (i, i * 0, i * 0)),
        ],
        out_specs=pl.BlockSpec((block_g, NMAX, EMB), lambda i: (i, i * 0, i * 0)),
        out_shape=jax.ShapeDtypeStruct((R, NMAX, EMB), jnp.float32),
        compiler_params=pltpu.CompilerParams(
            dimension_semantics=("parallel",),
        ),
    )(m, w, x)


def kernel(edge_dense, ring_index, batch, weight):
    B, NMAX, _, EMB = edge_dense.shape
    E = ring_index.shape[1]
    R = B * NMAX
    P = R * NMAX

    ring32 = ring_index.astype(jnp.int32)
    mask2 = _make_sc_mask(E, P, NMAX)(ring32)
    m = mask2.reshape(2, R, NMAX)

    x = edge_dense.reshape(R, NMAX, EMB)
    w = weight.astype(jnp.float32)
    out = _dense_add(x, m, w)
    return out.reshape(B, NMAX, NMAX, EMB)


# speedup vs baseline: 1.1658x; 1.0271x over previous
import functools

import jax
import jax.numpy as jnp
from jax import lax
from jax.experimental import pallas as pl
from jax.experimental.pallas import tpu as pltpu
from jax.experimental.pallas import tpu_sc as plsc

_NC = 2
_NS = 16
_L = 16


def _make_sc_mask(E, P, NMAX):
    e_w = E // (_NC * _NS)
    assert e_w % _L == 0 and e_w % 128 == 0
    n_chunk = e_w // 128
    p_s = P // _NS
    assert p_s % 8 == 0

    mesh = plsc.VectorSubcoreMesh(
        core_axis_name="c", subcore_axis_name="s",
        num_cores=_NC, num_subcores=_NS)

    @functools.partial(
        pl.kernel,
        out_type=jax.ShapeDtypeStruct((_NC, P), jnp.float32),
        mesh=mesh,
        scratch_types=[
            pltpu.VMEM((e_w,), jnp.int32),
            pltpu.VMEM((e_w,), jnp.int32),
            pltpu.VMEM((n_chunk, 128), jnp.int32),
            pltpu.VMEM((128,), jnp.float32),
            pltpu.VMEM((p_s,), jnp.float32),
            pltpu.VMEM_SHARED((P,), jnp.float32),
            pltpu.SemaphoreType.DMA,
            pltpu.SemaphoreType.DMA,
        ],
    )
    def sc_mask(ring_hbm, out_hbm, src_v, dst_v, idx_v, ones_v, zero_v,
                mask_sh, in_sem, sc_sem):
        c = lax.axis_index("c")
        s = lax.axis_index("s")
        base = (c * _NS + s) * e_w
        i32_0 = c * 0
        i32_1 = i32_0 + 1

        ld_src = pltpu.async_copy(ring_hbm.at[i32_0, pl.ds(base, e_w)], src_v, in_sem)
        ld_dst = pltpu.async_copy(ring_hbm.at[i32_1, pl.ds(base, e_w)], dst_v, in_sem)

        i32_stride = i32_0 + 64 * _L

        def zfill(i, carry):
            b = i * i32_stride
            for t in range(64):
                zero_v[pl.ds(b + t * _L, _L)] = jnp.zeros((_L,), jnp.float32)
            return carry
        lax.fori_loop(jnp.int32(0), jnp.int32(p_s // (64 * _L)), zfill, 0)

        for t in range(128 // _L):
            ones_v[pl.ds(t * _L, _L)] = jnp.full((_L,), 1.0, jnp.float32)

        zero_cp = pltpu.async_copy(zero_v, mask_sh.at[pl.ds(s * p_s, p_s)], sc_sem)

        ld_src.wait()
        ld_dst.wait()

        for j in range(n_chunk):
            for t in range(128 // _L):
                off = j * 128 + t * _L
                sv = src_v[pl.ds(off, _L)]
                dv = dst_v[pl.ds(off, _L)]
                idx_v[i32_0 + j, pl.ds(t * _L, _L)] = sv * NMAX + (dv & (NMAX - 1))

        zero_cp.wait()
        plsc.subcore_barrier()
        descs = [pltpu.async_copy(ones_v, mask_sh.at[idx_v.at[i32_0 + j]],
                                  sc_sem, add=True)
                 for j in range(n_chunk)]
        for d in descs:
            d.wait()
        plsc.subcore_barrier()
        pltpu.sync_copy(mask_sh.at[pl.ds(s * p_s, p_s)],
                        out_hbm.at[c, pl.ds(s * p_s, p_s)])

    return sc_mask


def _dense_body(m_ref, w_ref, x_ref, o_ref):
    m = m_ref[...]
    x = x_ref[...]
    w1 = w_ref[1, :]
    factor = jnp.minimum(m[0] + m[1], 1.0)
    o_ref[...] = x + factor[:, :, None] * w1[None, None, :]


def _dense_add(x, m, w, block_g=128):
    R, NMAX, EMB = x.shape
    grid = (R // block_g,)
    return pl.pallas_call(
        _dense_body,
        grid=grid,
        in_specs=[
            pl.BlockSpec((2, block_g, NMAX), lambda i: (i * 0, i, i * 0)),
            pl.BlockSpec((2, EMB), lambda i: (i * 0, i * 0)),
            pl.BlockSpec((block_g, NMAX, EMB), lambda i: (i, i * 0, i * 0)),
        ],
        out_specs=pl.BlockSpec((block_g, NMAX, EMB), lambda i: (i, i * 0, i * 0)),
        out_shape=jax.ShapeDtypeStruct((R, NMAX, EMB), jnp.float32),
        compiler_params=pltpu.CompilerParams(
            dimension_semantics=("parallel",),
        ),
    )(m, w, x)


def kernel(edge_dense, ring_index, batch, weight):
    B, NMAX, _, EMB = edge_dense.shape
    E = ring_index.shape[1]
    R = B * NMAX
    P = R * NMAX

    ring32 = ring_index.astype(jnp.int32)
    mask2 = _make_sc_mask(E, P, NMAX)(ring32)
    m = mask2.reshape(2, R, NMAX)

    x = edge_dense.reshape(R, NMAX, EMB)
    w = weight.astype(jnp.float32)
    out = _dense_add(x, m, w)
    return out.reshape(B, NMAX, NMAX, EMB)
